# speedup vs baseline: 1.2165x; 1.2165x over previous
import jax
import jax.numpy as jnp
from jax.experimental import pallas as pl
from jax.experimental.pallas import tpu as pltpu


def _rvq_body(x_ref, cb_ref, out_ref, codes_ref):
    r = x_ref[0]
    qt = jnp.zeros_like(r)
    Q, K, _ = cb_ref.shape
    for q in range(Q):
        cb = cb_ref[q]
        cbn = jnp.sum(cb * cb, axis=1)
        rn = jnp.sum(r * r, axis=0)
        scores = jax.lax.dot_general(
            cb, r, (((1,), (0,)), ((), ())),
            preferred_element_type=jnp.float32)
        dist = (rn[None, :] - 2.0 * scores) + cbn[:, None]
        mn = jnp.min(dist, axis=0)
        iota = jax.lax.broadcasted_iota(jnp.int32, dist.shape, 0)
        idx = jnp.min(jnp.where(dist == mn[None, :], iota, K), axis=0)
        oh = (iota == idx[None, :]).astype(jnp.float32)
        quant = jax.lax.dot_general(
            cb, oh, (((0,), (0,)), ((), ())),
            preferred_element_type=jnp.float32,
            precision=jax.lax.Precision.HIGHEST)
        r = r - quant
        qt = qt + quant
        codes_ref[0, 0, q, :] = idx
    out_ref[0] = qt


def kernel(embeddings, embed):
    B, D, T = embeddings.shape
    Q, K, _ = embed.shape
    Tb = 512
    grid = (B, T // Tb)
    qout, codes4 = pl.pallas_call(
        _rvq_body,
        grid=grid,
        in_specs=[
            pl.BlockSpec((1, D, Tb), lambda b, t: (b, 0, t)),
            pl.BlockSpec((Q, K, D), lambda b, t: (0, 0, 0)),
        ],
        out_specs=(
            pl.BlockSpec((1, D, Tb), lambda b, t: (b, 0, t)),
            pl.BlockSpec((1, 1, Q, Tb), lambda b, t: (b, t, 0, 0)),
        ),
        out_shape=(
            jax.ShapeDtypeStruct((B, D, T), jnp.float32),
            jax.ShapeDtypeStruct((B, T // Tb, Q, Tb), jnp.int32),
        ),
        compiler_params=pltpu.CompilerParams(
            dimension_semantics=("parallel", "parallel")),
    )(embeddings, embed)
    codes = jnp.transpose(codes4, (2, 0, 1, 3)).reshape(Q, B, T)
    return (qout, codes)


# speedup vs baseline: 1.9268x; 1.5839x over previous
import jax
import jax.numpy as jnp
from jax.experimental import pallas as pl
from jax.experimental.pallas import tpu as pltpu


def _rvq_body(x_ref, cb_ref, c1_ref, c2_ref, c3_ref, out_ref, codes_ref):
    r = x_ref[0]
    qt = jnp.zeros_like(r)
    Q, K, _ = cb_ref.shape
    for q in range(Q):
        cb = cb_ref[q]
        cbn = jnp.sum(cb * cb, axis=1)
        rn = jnp.sum(r * r, axis=0)
        scores = jax.lax.dot_general(
            cb, r, (((1,), (0,)), ((), ())),
            preferred_element_type=jnp.float32)
        dist = (rn[None, :] - 2.0 * scores) + cbn[:, None]
        mn = jnp.min(dist, axis=0)
        iota = jax.lax.broadcasted_iota(jnp.int32, dist.shape, 0)
        idx = jnp.min(jnp.where(dist == mn[None, :], iota, K), axis=0)
        oh = (iota == idx[None, :]).astype(jnp.bfloat16)
        dn = (((0,), (0,)), ((), ()))
        d1 = jax.lax.dot_general(c1_ref[q], oh, dn,
                                 preferred_element_type=jnp.float32)
        d2 = jax.lax.dot_general(c2_ref[q], oh, dn,
                                 preferred_element_type=jnp.float32)
        d3 = jax.lax.dot_general(c3_ref[q], oh, dn,
                                 preferred_element_type=jnp.float32)
        quant = (d1 + d2) + d3
        r = r - quant
        qt = qt + quant
        codes_ref[0, 0, q, :] = idx
    out_ref[0] = qt


def _split_body(e_ref, c1_ref, c2_ref, c3_ref):
    e = e_ref[...]
    c1 = e.astype(jnp.bfloat16)
    r1 = e - c1.astype(jnp.float32)
    c2 = r1.astype(jnp.bfloat16)
    r2 = r1 - c2.astype(jnp.float32)
    c1_ref[...] = c1
    c2_ref[...] = c2
    c3_ref[...] = r2.astype(jnp.bfloat16)


def kernel(embeddings, embed):
    B, D, T = embeddings.shape
    Q, K, _ = embed.shape
    c1, c2, c3 = pl.pallas_call(
        _split_body,
        grid=(Q,),
        in_specs=[pl.BlockSpec((1, K, D), lambda q: (q, 0, 0))],
        out_specs=tuple(pl.BlockSpec((1, K, D), lambda q: (q, 0, 0))
                        for _ in range(3)),
        out_shape=tuple(jax.ShapeDtypeStruct((Q, K, D), jnp.bfloat16)
                        for _ in range(3)),
    )(embed)
    Tb = 512
    grid = (B, T // Tb)
    cb_spec = pl.BlockSpec((Q, K, D), lambda b, t: (0, 0, 0))
    qout, codes4 = pl.pallas_call(
        _rvq_body,
        grid=grid,
        in_specs=[
            pl.BlockSpec((1, D, Tb), lambda b, t: (b, 0, t)),
            cb_spec, cb_spec, cb_spec, cb_spec,
        ],
        out_specs=(
            pl.BlockSpec((1, D, Tb), lambda b, t: (b, 0, t)),
            pl.BlockSpec((1, 1, Q, Tb), lambda b, t: (b, t, 0, 0)),
        ),
        out_shape=(
            jax.ShapeDtypeStruct((B, D, T), jnp.float32),
            jax.ShapeDtypeStruct((B, T // Tb, Q, Tb), jnp.int32),
        ),
        compiler_params=pltpu.CompilerParams(
            dimension_semantics=("parallel", "parallel")),
    )(embeddings, embed, c1, c2, c3)
    codes = jnp.transpose(codes4, (2, 0, 1, 3)).reshape(Q, B, T)
    return (qout, codes)


# speedup vs baseline: 2.5819x; 1.3400x over previous
import jax
import jax.numpy as jnp
from jax.experimental import pallas as pl
from jax.experimental.pallas import tpu as pltpu


def _split_body(e_ref, c1_ref, c2_ref, c3_ref, cbn_ref):
    e = e_ref[...]
    c1 = e.astype(jnp.bfloat16)
    r1 = e - c1.astype(jnp.float32)
    c2 = r1.astype(jnp.bfloat16)
    r2 = r1 - c2.astype(jnp.float32)
    c1_ref[...] = c1
    c2_ref[...] = c2
    c3_ref[...] = r2.astype(jnp.bfloat16)
    cb = e[0]
    cbn = jnp.sum(cb * cb, axis=1)
    cbn_ref[0] = jnp.broadcast_to(cbn[:, None], cbn_ref.shape[1:])


def _rvq_body(x_ref, cb_ref, c1_ref, c2_ref, c3_ref, cbn_ref,
              out_ref, codes_ref):
    r = x_ref[0]
    qt = jnp.zeros_like(r)
    Q, K, _ = cb_ref.shape
    Tb = r.shape[1]
    for q in range(Q):
        cb = cb_ref[q]
        cbn = cbn_ref[q][:, :1]
        rn = jnp.sum(r * r, axis=0)
        scores = jax.lax.dot_general(
            cb, r, (((1,), (0,)), ((), ())),
            preferred_element_type=jnp.float32)
        dist = (rn[None, :] - 2.0 * scores) + cbn
        idx = jnp.argmin(dist, axis=0)
        iota = jax.lax.broadcasted_iota(jnp.int32, (K, Tb), 0)
        oh = (iota == idx[None, :]).astype(jnp.bfloat16)
        dn = (((0,), (0,)), ((), ()))
        d1 = jax.lax.dot_general(c1_ref[q], oh, dn,
                                 preferred_element_type=jnp.float32)
        d2 = jax.lax.dot_general(c2_ref[q], oh, dn,
                                 preferred_element_type=jnp.float32)
        d3 = jax.lax.dot_general(c3_ref[q], oh, dn,
                                 preferred_element_type=jnp.float32)
        quant = (d1 + d2) + d3
        r = r - quant
        qt = qt + quant
        codes_ref[0, 0, q, :] = idx
    out_ref[0] = qt


def kernel(embeddings, embed):
    B, D, T = embeddings.shape
    Q, K, _ = embed.shape
    c1, c2, c3, cbn = pl.pallas_call(
        _split_body,
        grid=(Q,),
        in_specs=[pl.BlockSpec((1, K, D), lambda q: (q, 0, 0))],
        out_specs=(
            pl.BlockSpec((1, K, D), lambda q: (q, 0, 0)),
            pl.BlockSpec((1, K, D), lambda q: (q, 0, 0)),
            pl.BlockSpec((1, K, D), lambda q: (q, 0, 0)),
            pl.BlockSpec((1, K, 128), lambda q: (q, 0, 0)),
        ),
        out_shape=(
            jax.ShapeDtypeStruct((Q, K, D), jnp.bfloat16),
            jax.ShapeDtypeStruct((Q, K, D), jnp.bfloat16),
            jax.ShapeDtypeStruct((Q, K, D), jnp.bfloat16),
            jax.ShapeDtypeStruct((Q, K, 128), jnp.float32),
        ),
    )(embed)
    Tb = 1024
    grid = (B, T // Tb)
    cb_spec = pl.BlockSpec((Q, K, D), lambda b, t: (0, 0, 0))
    qout, codes4 = pl.pallas_call(
        _rvq_body,
        grid=grid,
        in_specs=[
            pl.BlockSpec((1, D, Tb), lambda b, t: (b, 0, t)),
            cb_spec, cb_spec, cb_spec, cb_spec,
            pl.BlockSpec((Q, K, 128), lambda b, t: (0, 0, 0)),
        ],
        out_specs=(
            pl.BlockSpec((1, D, Tb), lambda b, t: (b, 0, t)),
            pl.BlockSpec((1, 1, Q, Tb), lambda b, t: (b, t, 0, 0)),
        ),
        out_shape=(
            jax.ShapeDtypeStruct((B, D, T), jnp.float32),
            jax.ShapeDtypeStruct((B, T // Tb, Q, Tb), jnp.int32),
        ),
        compiler_params=pltpu.CompilerParams(
            dimension_semantics=("parallel", "parallel")),
    )(embeddings, embed, c1, c2, c3, cbn)
    codes = jnp.transpose(codes4, (2, 0, 1, 3)).reshape(Q, B, T)
    return (qout, codes)
